# baseline (speedup 1.0000x reference)
import jax
import jax.numpy as jnp
from jax.experimental import pallas as pl
from jax.experimental.pallas import tpu as pltpu

TOKENS = 32768
D = 1024
E = 8
BT = 2048
NBLK = TOKENS // BT
NBUF = 4


def _gating_kernel(x_hbm, wt_ref, b_ref, o1_hbm, o2_hbm, buf, sems, obuf, osems):
    wt = wt_ref[...]
    b = b_ref[...]

    def in_copy(i):
        slot = jax.lax.rem(i, NBUF)
        return pltpu.make_async_copy(
            x_hbm.at[pl.ds(i * BT, BT), :], buf.at[slot], sems.at[slot]
        )

    def out_copy(i, which, o_hbm):
        slot = jax.lax.rem(i, 2)
        return pltpu.make_async_copy(
            obuf.at[slot], o_hbm.at[pl.ds(i * BT, BT), :], osems.at[which, slot]
        )

    for j in range(NBUF - 1):
        in_copy(j).start()

    def body(i, _):
        @pl.when(i + NBUF - 1 < NBLK)
        def _():
            in_copy(i + NBUF - 1).start()

        @pl.when(i >= 2)
        def _():
            out_copy(i - 2, 0, o1_hbm).wait()
            out_copy(i - 2, 1, o2_hbm).wait()

        in_copy(i).wait()
        slot = jax.lax.rem(i, NBUF)
        oslot = jax.lax.rem(i, 2)
        obuf[oslot] = (
            jnp.dot(buf[slot], wt, preferred_element_type=jnp.float32) + b
        )
        out_copy(i, 0, o1_hbm).start()
        out_copy(i, 1, o2_hbm).start()
        return ()

    jax.lax.fori_loop(0, NBLK, body, (), unroll=2)

    for j in range(NBLK - 2, NBLK):
        out_copy(j, 0, o1_hbm).wait()
        out_copy(j, 1, o2_hbm).wait()


def kernel(x, W, b, train):
    wt = W.T
    b2 = b.reshape(1, E)
    gates, gates2 = pl.pallas_call(
        _gating_kernel,
        in_specs=[
            pl.BlockSpec(memory_space=pltpu.MemorySpace.HBM),
            pl.BlockSpec(memory_space=pltpu.VMEM),
            pl.BlockSpec(memory_space=pltpu.VMEM),
        ],
        out_specs=[
            pl.BlockSpec(memory_space=pltpu.MemorySpace.HBM),
            pl.BlockSpec(memory_space=pltpu.MemorySpace.HBM),
        ],
        out_shape=[
            jax.ShapeDtypeStruct((TOKENS, E), jnp.float32),
            jax.ShapeDtypeStruct((TOKENS, E), jnp.float32),
        ],
        scratch_shapes=[
            pltpu.VMEM((NBUF, BT, D), jnp.float32),
            pltpu.SemaphoreType.DMA((NBUF,)),
            pltpu.VMEM((2, BT, E), jnp.float32),
            pltpu.SemaphoreType.DMA((2, 2)),
        ],
        compiler_params=pltpu.CompilerParams(
            vmem_limit_bytes=48 * 1024 * 1024,
        ),
    )(x, wt, b2)
    return (gates, gates2)


# speedup vs baseline: 1.6201x; 1.6201x over previous
import jax
import jax.numpy as jnp
from jax.experimental import pallas as pl
from jax.experimental.pallas import tpu as pltpu

TOKENS = 32768
D = 1024
E = 8
BT = 2048
NBLK = TOKENS // BT
NBUF = 4


def _gating_kernel(x_hbm, w_ref, b_ref, o1_hbm, o2_hbm, buf, sems, obuf, osems):
    w = w_ref[...]
    b = b_ref[...]

    def in_copy(i):
        slot = jax.lax.rem(i, NBUF)
        return pltpu.make_async_copy(
            x_hbm.at[pl.ds(i * BT, BT), :], buf.at[slot], sems.at[slot]
        )

    def out_copy(i, which, o_hbm):
        slot = jax.lax.rem(i, 2)
        return pltpu.make_async_copy(
            obuf.at[slot], o_hbm.at[:, pl.ds(i * BT, BT)], osems.at[which, slot]
        )

    for j in range(NBUF - 1):
        in_copy(j).start()

    def body(i, _):
        @pl.when(i + NBUF - 1 < NBLK)
        def _():
            in_copy(i + NBUF - 1).start()

        @pl.when(i >= 2)
        def _():
            out_copy(i - 2, 0, o1_hbm).wait()
            out_copy(i - 2, 1, o2_hbm).wait()

        in_copy(i).wait()
        slot = jax.lax.rem(i, NBUF)
        oslot = jax.lax.rem(i, 2)
        obuf[oslot] = (
            jax.lax.dot_general(
                w,
                buf[slot],
                (((1,), (1,)), ((), ())),
                preferred_element_type=jnp.float32,
            )
            + b
        )
        out_copy(i, 0, o1_hbm).start()
        out_copy(i, 1, o2_hbm).start()
        return ()

    jax.lax.fori_loop(0, NBLK, body, (), unroll=2)

    for j in range(NBLK - 2, NBLK):
        out_copy(j, 0, o1_hbm).wait()
        out_copy(j, 1, o2_hbm).wait()


def kernel(x, W, b, train):
    b2 = b.reshape(E, 1)
    gt1, gt2 = pl.pallas_call(
        _gating_kernel,
        in_specs=[
            pl.BlockSpec(memory_space=pltpu.MemorySpace.HBM),
            pl.BlockSpec(memory_space=pltpu.VMEM),
            pl.BlockSpec(memory_space=pltpu.VMEM),
        ],
        out_specs=[
            pl.BlockSpec(memory_space=pltpu.MemorySpace.HBM),
            pl.BlockSpec(memory_space=pltpu.MemorySpace.HBM),
        ],
        out_shape=[
            jax.ShapeDtypeStruct((E, TOKENS), jnp.float32),
            jax.ShapeDtypeStruct((E, TOKENS), jnp.float32),
        ],
        scratch_shapes=[
            pltpu.VMEM((NBUF, BT, D), jnp.float32),
            pltpu.SemaphoreType.DMA((NBUF,)),
            pltpu.VMEM((2, E, BT), jnp.float32),
            pltpu.SemaphoreType.DMA((2, 2)),
        ],
        compiler_params=pltpu.CompilerParams(
            vmem_limit_bytes=48 * 1024 * 1024,
        ),
    )(x, W, b2)
    return (gt1.T, gt2.T)


# speedup vs baseline: 1.6921x; 1.0444x over previous
import jax
import jax.numpy as jnp
from jax.experimental import pallas as pl
from jax.experimental.pallas import tpu as pltpu

TOKENS = 32768
D = 1024
E = 8
BT = 2048


def _gating_kernel(x_ref, w_ref, b_ref, o1_ref, o2_ref):
    g = (
        jax.lax.dot_general(
            w_ref[...],
            x_ref[...],
            (((1,), (1,)), ((), ())),
            preferred_element_type=jnp.float32,
        )
        + b_ref[...]
    )
    o1_ref[...] = g
    o2_ref[...] = g


def kernel(x, W, b, train):
    b2 = b.reshape(E, 1)
    gt1, gt2 = pl.pallas_call(
        _gating_kernel,
        grid=(TOKENS // BT,),
        in_specs=[
            pl.BlockSpec((BT, D), lambda i: (i, 0)),
            pl.BlockSpec((E, D), lambda i: (0, 0)),
            pl.BlockSpec((E, 1), lambda i: (0, 0)),
        ],
        out_specs=[
            pl.BlockSpec((E, BT), lambda i: (0, i)),
            pl.BlockSpec((E, BT), lambda i: (0, i)),
        ],
        out_shape=[
            jax.ShapeDtypeStruct((E, TOKENS), jnp.float32),
            jax.ShapeDtypeStruct((E, TOKENS), jnp.float32),
        ],
        compiler_params=pltpu.CompilerParams(
            dimension_semantics=("parallel",),
        ),
    )(x, W, b2)
    return (gt1.T, gt2.T)
